# baseline (speedup 1.0000x reference)
import jax
import jax.numpy as jnp
from jax import lax
from jax.experimental import pallas as pl
from jax.experimental.pallas import tpu as pltpu
from jax.experimental.pallas import tpu_sc as plsc

PERM = (2, 3, 0, 1, 4, 6, 5, 7, 9, 8, 11, 10, 12, 14, 13, 16, 15, 18, 17)
REV = (4, 7, 12)
B, P, H, W = 16, 19, 48, 48
L = 16
NC = W // L
NSLOT = 3

_mesh = plsc.VectorSubcoreMesh(
    core_axis_name="c", subcore_axis_name="s", num_cores=2, num_subcores=16
)


def _worker_id():
    return lax.axis_index("s") * 2 + lax.axis_index("c")


def _src_p(p):
    psrc = jnp.int32(PERM[0])
    for k in range(1, P):
        psrc = jnp.where(p == k, PERM[k], psrc)
    return psrc


def _is_rev(p):
    return (p == REV[0]) | (p == REV[1]) | (p == REV[2])


def _sc_body(f0, f1, f2, o0, o2, in0, in2, ob0, ob2, sin, sout):
    wid = _worker_id()
    b = wid % 16
    group = wid // 16
    p_lo = jnp.where(group == 0, 0, 10)
    p_hi = jnp.where(group == 0, 10, 19)

    def start_in(s, p):
        sp = _src_p(p)
        pltpu.async_copy(f0.at[b, sp], in0[s], sin[s])

        @pl.when(_is_rev(p))
        def _():
            pltpu.async_copy(f1.at[b, p], in2[s], sin[s])

        @pl.when(jnp.logical_not(_is_rev(p)))
        def _():
            pltpu.async_copy(f2.at[b, sp], in2[s], sin[s])

    def wait_in(s, p):
        sp = _src_p(p)
        pltpu.make_async_copy(f0.at[b, sp], in0[s], sin[s]).wait()
        pltpu.make_async_copy(f2.at[b, sp], in2[s], sin[s]).wait()

    def start_out(s, p):
        pltpu.async_copy(ob0[s], o0.at[b, p], sout[s])
        pltpu.async_copy(ob2[s], o2.at[b, p], sout[s])

    def wait_out(s, p):
        pltpu.make_async_copy(ob0[s], o0.at[b, p], sout[s]).wait()
        pltpu.make_async_copy(ob2[s], o2.at[b, p], sout[s]).wait()

    for s in range(NSLOT):
        start_in(s, p_lo + s)

    @pl.loop(0, 12, step=NSLOT)
    def _iter(j):
        for s in range(NSLOT):
            idx = j + s
            p = p_lo + idx

            @pl.when(p < p_hi)
            def _():
                wait_in(s, p)

                @pl.when(idx >= NSLOT)
                def _():
                    wait_out(s, p - NSLOT)

                @plsc.parallel_loop(0, H, unroll=8)
                def _r0(r):
                    for c in range(NC):
                        v = in0[s][r, pl.ds(c * L, L)]
                        ob0[s][r, pl.ds((NC - 1 - c) * L, L)] = jnp.flip(v)

                @plsc.parallel_loop(0, H, unroll=8)
                def _rneg(r):
                    for c in range(NC):
                        v = in2[s][0, r, pl.ds(c * L, L)]
                        ob2[s][0, r, pl.ds((NC - 1 - c) * L, L)] = -jnp.flip(v)

                @plsc.parallel_loop(0, H, unroll=8)
                def _rpos(r):
                    for c in range(NC):
                        v = in2[s][1, r, pl.ds(c * L, L)]
                        ob2[s][1, r, pl.ds((NC - 1 - c) * L, L)] = jnp.flip(v)

                start_out(s, p)

                @pl.when(p + NSLOT < p_hi)
                def _():
                    start_in(s, p + NSLOT)

    for s in range(NSLOT):
        wait_out(s, p_lo + s)


_sc_call = pl.kernel(
    _sc_body,
    out_type=(
        jax.ShapeDtypeStruct((B, P, H, W), jnp.float32),
        jax.ShapeDtypeStruct((B, P, 2, H, W), jnp.float32),
    ),
    mesh=_mesh,
    scratch_types=(
        [pltpu.VMEM((H, W), jnp.float32) for _ in range(NSLOT)],
        [pltpu.VMEM((2, H, W), jnp.float32) for _ in range(NSLOT)],
        [pltpu.VMEM((H, W), jnp.float32) for _ in range(NSLOT)],
        [pltpu.VMEM((2, H, W), jnp.float32) for _ in range(NSLOT)],
        [pltpu.SemaphoreType.DMA for _ in range(NSLOT)],
        [pltpu.SemaphoreType.DMA for _ in range(NSLOT)],
    ),
)


def _tc_body(f1_ref, f2a_ref, f2b_ref, f2c_ref, o1_ref):
    idx1 = W - 1 - lax.broadcasted_iota(jnp.int32, (2, H, W), 2)
    revmap = {REV[0]: f2a_ref, REV[1]: f2b_ref, REV[2]: f2c_ref}
    for p in range(P):
        src = revmap[p][0, 0] if p in REV else f1_ref[0, PERM[p]]
        y = jnp.take_along_axis(src, idx1, axis=2)
        o1_ref[0, p, 0] = -y[0]
        o1_ref[0, p, 1] = y[1]


_tc_call = pl.pallas_call(
    _tc_body,
    grid=(B,),
    in_specs=[
        pl.BlockSpec((1, P, 2, H, W), lambda b: (b, 0, 0, 0, 0)),
        pl.BlockSpec((1, 1, 2, H, W), lambda b: (b, REV[0], 0, 0, 0)),
        pl.BlockSpec((1, 1, 2, H, W), lambda b: (b, REV[1], 0, 0, 0)),
        pl.BlockSpec((1, 1, 2, H, W), lambda b: (b, REV[2], 0, 0, 0)),
    ],
    out_specs=pl.BlockSpec((1, P, 2, H, W), lambda b: (b, 0, 0, 0, 0)),
    out_shape=jax.ShapeDtypeStruct((B, P, 2, H, W), jnp.float32),
)


@jax.jit
def kernel(field0, field1, field2):
    o0, o2 = _sc_call(field0, field1, field2)
    o1 = _tc_call(field1, field2, field2, field2)
    return (o0, o1, o2)


# speedup vs baseline: 1.0144x; 1.0144x over previous
import jax
import jax.numpy as jnp
from jax import lax
from jax.experimental import pallas as pl
from jax.experimental.pallas import tpu as pltpu
from jax.experimental.pallas import tpu_sc as plsc

PERM = (2, 3, 0, 1, 4, 6, 5, 7, 9, 8, 11, 10, 12, 14, 13, 16, 15, 18, 17)
REV = (4, 7, 12)
B, P, H, W = 16, 19, 48, 48
L = 16
NC = W // L
NSLOT = 3

_mesh = plsc.VectorSubcoreMesh(
    core_axis_name="c", subcore_axis_name="s", num_cores=2, num_subcores=16
)


def _worker_id():
    return lax.axis_index("s") * 2 + lax.axis_index("c")


def _src_p(p):
    psrc = jnp.int32(PERM[0])
    for k in range(1, P):
        psrc = jnp.where(p == k, PERM[k], psrc)
    return psrc


def _is_rev(p):
    return (p == REV[0]) | (p == REV[1]) | (p == REV[2])


def _sc_body(f0, f1, f2, o0, o2, in0, in2, ob0, ob2, sin, sout):
    wid = _worker_id()
    b = wid % 16
    group = wid // 16
    p_lo = jnp.where(group == 0, 0, 10)
    p_hi = jnp.where(group == 0, 10, 19)

    def start_in(s, p):
        sp = _src_p(p)
        pltpu.async_copy(f0.at[b, sp], in0[s], sin[s])

        @pl.when(_is_rev(p))
        def _():
            pltpu.async_copy(f1.at[b, p], in2[s], sin[s])

        @pl.when(jnp.logical_not(_is_rev(p)))
        def _():
            pltpu.async_copy(f2.at[b, sp], in2[s], sin[s])

    def wait_in(s, p):
        sp = _src_p(p)
        pltpu.make_async_copy(f0.at[b, sp], in0[s], sin[s]).wait()
        pltpu.make_async_copy(f2.at[b, sp], in2[s], sin[s]).wait()

    def start_out(s, p):
        pltpu.async_copy(ob0[s], o0.at[b, p], sout[s])
        pltpu.async_copy(ob2[s], o2.at[b, p], sout[s])

    def wait_out(s, p):
        pltpu.make_async_copy(ob0[s], o0.at[b, p], sout[s]).wait()
        pltpu.make_async_copy(ob2[s], o2.at[b, p], sout[s]).wait()

    for s in range(NSLOT):
        start_in(s, p_lo + s)

    @pl.loop(0, 12, step=NSLOT)
    def _iter(j):
        for s in range(NSLOT):
            idx = j + s
            p = p_lo + idx

            @pl.when(p < p_hi)
            def _():
                wait_in(s, p)

                @pl.when(idx >= NSLOT)
                def _():
                    wait_out(s, p - NSLOT)

                @plsc.parallel_loop(0, H, unroll=8)
                def _r0(r):
                    for c in range(NC):
                        v = in0[s][r, pl.ds(c * L, L)]
                        ob0[s][r, pl.ds((NC - 1 - c) * L, L)] = jnp.flip(v)

                @plsc.parallel_loop(0, H, unroll=8)
                def _rneg(r):
                    for c in range(NC):
                        v = in2[s][0, r, pl.ds(c * L, L)]
                        ob2[s][0, r, pl.ds((NC - 1 - c) * L, L)] = -jnp.flip(v)

                @plsc.parallel_loop(0, H, unroll=8)
                def _rpos(r):
                    for c in range(NC):
                        v = in2[s][1, r, pl.ds(c * L, L)]
                        ob2[s][1, r, pl.ds((NC - 1 - c) * L, L)] = jnp.flip(v)

                start_out(s, p)

                @pl.when(p + NSLOT < p_hi)
                def _():
                    start_in(s, p + NSLOT)

    for s in range(NSLOT):
        wait_out(s, p_lo + s)


_sc_call = pl.kernel(
    _sc_body,
    out_type=(
        jax.ShapeDtypeStruct((B, P, H, W), jnp.float32),
        jax.ShapeDtypeStruct((B, P, 2, H, W), jnp.float32),
    ),
    mesh=_mesh,
    scratch_types=(
        [pltpu.VMEM((H, W), jnp.float32) for _ in range(NSLOT)],
        [pltpu.VMEM((2, H, W), jnp.float32) for _ in range(NSLOT)],
        [pltpu.VMEM((H, W), jnp.float32) for _ in range(NSLOT)],
        [pltpu.VMEM((2, H, W), jnp.float32) for _ in range(NSLOT)],
        [pltpu.SemaphoreType.DMA for _ in range(NSLOT)],
        [pltpu.SemaphoreType.DMA for _ in range(NSLOT)],
    ),
)


TCB = 2


def _tc_body(f1_ref, f2a_ref, f2b_ref, f2c_ref, o1_ref):
    idx1 = W - 1 - lax.broadcasted_iota(jnp.int32, (2, H, W), 2)
    revmap = {REV[0]: f2a_ref, REV[1]: f2b_ref, REV[2]: f2c_ref}
    for bi in range(TCB):
        for p in range(P):
            src = revmap[p][bi, 0] if p in REV else f1_ref[bi, PERM[p]]
            y = jnp.take_along_axis(src, idx1, axis=2)
            o1_ref[bi, p, 0] = -y[0]
            o1_ref[bi, p, 1] = y[1]


_tc_call = pl.pallas_call(
    _tc_body,
    grid=(B // TCB,),
    in_specs=[
        pl.BlockSpec((TCB, P, 2, H, W), lambda b: (b, 0, 0, 0, 0)),
        pl.BlockSpec((TCB, 1, 2, H, W), lambda b: (b, REV[0], 0, 0, 0)),
        pl.BlockSpec((TCB, 1, 2, H, W), lambda b: (b, REV[1], 0, 0, 0)),
        pl.BlockSpec((TCB, 1, 2, H, W), lambda b: (b, REV[2], 0, 0, 0)),
    ],
    out_specs=pl.BlockSpec((TCB, P, 2, H, W), lambda b: (b, 0, 0, 0, 0)),
    out_shape=jax.ShapeDtypeStruct((B, P, 2, H, W), jnp.float32),
)


@jax.jit
def kernel(field0, field1, field2):
    o0, o2 = _sc_call(field0, field1, field2)
    o1 = _tc_call(field1, field2, field2, field2)
    return (o0, o1, o2)


# speedup vs baseline: 1.0408x; 1.0260x over previous
import jax
import jax.numpy as jnp
from jax import lax
from jax.experimental import pallas as pl
from jax.experimental.pallas import tpu as pltpu
from jax.experimental.pallas import tpu_sc as plsc

PERM = (2, 3, 0, 1, 4, 6, 5, 7, 9, 8, 11, 10, 12, 14, 13, 16, 15, 18, 17)
REV = (4, 7, 12)
B, P, H, W = 16, 19, 48, 48
L = 16
NC = W // L
NSLOT = 3

_mesh = plsc.VectorSubcoreMesh(
    core_axis_name="c", subcore_axis_name="s", num_cores=2, num_subcores=16
)


def _worker_id():
    return lax.axis_index("s") * 2 + lax.axis_index("c")


def _src_p(p):
    psrc = jnp.int32(PERM[0])
    for k in range(1, P):
        psrc = jnp.where(p == k, PERM[k], psrc)
    return psrc


def _is_rev(p):
    return (p == REV[0]) | (p == REV[1]) | (p == REV[2])


def _sc_body(f0, f1, f2, o0, o2, in0, in2, ob0, ob2, sin, sout):
    wid = _worker_id()
    b = wid % 16
    group = wid // 16
    p_lo = jnp.where(group == 0, 0, 10)
    p_hi = jnp.where(group == 0, 10, 19)

    def start_in(s, p):
        sp = _src_p(p)
        pltpu.async_copy(f0.at[b, sp], in0[s], sin[s])

        @pl.when(_is_rev(p))
        def _():
            pltpu.async_copy(f1.at[b, p], in2[s], sin[s])

        @pl.when(jnp.logical_not(_is_rev(p)))
        def _():
            pltpu.async_copy(f2.at[b, sp], in2[s], sin[s])

    def wait_in(s, p):
        sp = _src_p(p)
        pltpu.make_async_copy(f0.at[b, sp], in0[s], sin[s]).wait()
        pltpu.make_async_copy(f2.at[b, sp], in2[s], sin[s]).wait()

    def start_out(s, p):
        pltpu.async_copy(ob0[s], o0.at[b, p], sout[s])
        pltpu.async_copy(ob2[s], o2.at[b, p], sout[s])

    def wait_out(s, p):
        pltpu.make_async_copy(ob0[s], o0.at[b, p], sout[s]).wait()
        pltpu.make_async_copy(ob2[s], o2.at[b, p], sout[s]).wait()

    for s in range(NSLOT):
        start_in(s, p_lo + s)

    @pl.loop(0, 12, step=NSLOT)
    def _iter(j):
        for s in range(NSLOT):
            idx = j + s
            p = p_lo + idx

            @pl.when(p < p_hi)
            def _():
                wait_in(s, p)

                @pl.when(idx >= NSLOT)
                def _():
                    wait_out(s, p - NSLOT)

                @plsc.parallel_loop(0, H, unroll=8)
                def _r0(r):
                    for c in range(NC):
                        v = in0[s][r, pl.ds(c * L, L)]
                        ob0[s][r, pl.ds((NC - 1 - c) * L, L)] = jnp.flip(v)

                @plsc.parallel_loop(0, H, unroll=8)
                def _rneg(r):
                    for c in range(NC):
                        v = in2[s][0, r, pl.ds(c * L, L)]
                        ob2[s][0, r, pl.ds((NC - 1 - c) * L, L)] = -jnp.flip(v)

                @plsc.parallel_loop(0, H, unroll=8)
                def _rpos(r):
                    for c in range(NC):
                        v = in2[s][1, r, pl.ds(c * L, L)]
                        ob2[s][1, r, pl.ds((NC - 1 - c) * L, L)] = jnp.flip(v)

                start_out(s, p)

                @pl.when(p + NSLOT < p_hi)
                def _():
                    start_in(s, p + NSLOT)

    for s in range(NSLOT):
        wait_out(s, p_lo + s)


_sc_call = pl.kernel(
    _sc_body,
    out_type=(
        jax.ShapeDtypeStruct((B, P, H, W), jnp.float32),
        jax.ShapeDtypeStruct((B, P, 2, H, W), jnp.float32),
    ),
    mesh=_mesh,
    scratch_types=(
        [pltpu.VMEM((H, W), jnp.float32) for _ in range(NSLOT)],
        [pltpu.VMEM((2, H, W), jnp.float32) for _ in range(NSLOT)],
        [pltpu.VMEM((H, W), jnp.float32) for _ in range(NSLOT)],
        [pltpu.VMEM((2, H, W), jnp.float32) for _ in range(NSLOT)],
        [pltpu.SemaphoreType.DMA for _ in range(NSLOT)],
        [pltpu.SemaphoreType.DMA for _ in range(NSLOT)],
    ),
)


TCB = 4


def _tc_body(f1_ref, f2a_ref, f2b_ref, f2c_ref, o1_ref):
    idx1 = W - 1 - lax.broadcasted_iota(jnp.int32, (2, H, W), 2)
    revmap = {REV[0]: f2a_ref, REV[1]: f2b_ref, REV[2]: f2c_ref}
    for bi in range(TCB):
        for p in range(P):
            src = revmap[p][bi, 0] if p in REV else f1_ref[bi, PERM[p]]
            y = jnp.take_along_axis(src, idx1, axis=2)
            o1_ref[bi, p, 0] = -y[0]
            o1_ref[bi, p, 1] = y[1]


_tc_call = pl.pallas_call(
    _tc_body,
    grid=(B // TCB,),
    in_specs=[
        pl.BlockSpec((TCB, P, 2, H, W), lambda b: (b, 0, 0, 0, 0)),
        pl.BlockSpec((TCB, 1, 2, H, W), lambda b: (b, REV[0], 0, 0, 0)),
        pl.BlockSpec((TCB, 1, 2, H, W), lambda b: (b, REV[1], 0, 0, 0)),
        pl.BlockSpec((TCB, 1, 2, H, W), lambda b: (b, REV[2], 0, 0, 0)),
    ],
    out_specs=pl.BlockSpec((TCB, P, 2, H, W), lambda b: (b, 0, 0, 0, 0)),
    out_shape=jax.ShapeDtypeStruct((B, P, 2, H, W), jnp.float32),
)


@jax.jit
def kernel(field0, field1, field2):
    o0, o2 = _sc_call(field0, field1, field2)
    o1 = _tc_call(field1, field2, field2, field2)
    return (o0, o1, o2)


# speedup vs baseline: 1.0457x; 1.0046x over previous
import jax
import jax.numpy as jnp
from jax import lax
from jax.experimental import pallas as pl
from jax.experimental.pallas import tpu as pltpu
from jax.experimental.pallas import tpu_sc as plsc

PERM = (2, 3, 0, 1, 4, 6, 5, 7, 9, 8, 11, 10, 12, 14, 13, 16, 15, 18, 17)
REV = (4, 7, 12)
B, P, H, W = 16, 19, 48, 48
L = 16
NC = W // L
NSLOT = 3

_mesh = plsc.VectorSubcoreMesh(
    core_axis_name="c", subcore_axis_name="s", num_cores=2, num_subcores=16
)


def _worker_id():
    return lax.axis_index("s") * 2 + lax.axis_index("c")


def _src_p(p):
    psrc = jnp.int32(PERM[0])
    for k in range(1, P):
        psrc = jnp.where(p == k, PERM[k], psrc)
    return psrc


def _is_rev(p):
    return (p == REV[0]) | (p == REV[1]) | (p == REV[2])


def _sc_body(f0, f1, f2, o0, o2, in0, in2, ob0, ob2, sin, sout):
    wid = _worker_id()
    b = wid % 16
    group = wid // 16
    p_lo = jnp.where(group == 0, 0, 10)
    p_hi = jnp.where(group == 0, 10, 19)

    def start_in(s, p):
        sp = _src_p(p)
        pltpu.async_copy(f0.at[b, sp], in0[s], sin[s])

        @pl.when(_is_rev(p))
        def _():
            pltpu.async_copy(f1.at[b, p], in2[s], sin[s])

        @pl.when(jnp.logical_not(_is_rev(p)))
        def _():
            pltpu.async_copy(f2.at[b, sp], in2[s], sin[s])

    def wait_in(s, p):
        sp = _src_p(p)
        pltpu.make_async_copy(f0.at[b, sp], in0[s], sin[s]).wait()
        pltpu.make_async_copy(f2.at[b, sp], in2[s], sin[s]).wait()

    def start_out(s, p):
        pltpu.async_copy(ob0[s], o0.at[b, p], sout[s])
        pltpu.async_copy(ob2[s], o2.at[b, p], sout[s])

    def wait_out(s, p):
        pltpu.make_async_copy(ob0[s], o0.at[b, p], sout[s]).wait()
        pltpu.make_async_copy(ob2[s], o2.at[b, p], sout[s]).wait()

    for s in range(NSLOT):
        start_in(s, p_lo + s)

    @pl.loop(0, 12, step=NSLOT)
    def _iter(j):
        for s in range(NSLOT):
            idx = j + s
            p = p_lo + idx

            @pl.when(p < p_hi)
            def _():
                wait_in(s, p)

                @pl.when(idx >= NSLOT)
                def _():
                    wait_out(s, p - NSLOT)

                @plsc.parallel_loop(0, H, unroll=8)
                def _r0(r):
                    for c in range(NC):
                        v = in0[s][r, pl.ds(c * L, L)]
                        ob0[s][r, pl.ds((NC - 1 - c) * L, L)] = jnp.flip(v)

                @plsc.parallel_loop(0, H, unroll=8)
                def _rneg(r):
                    for c in range(NC):
                        v = in2[s][0, r, pl.ds(c * L, L)]
                        ob2[s][0, r, pl.ds((NC - 1 - c) * L, L)] = -jnp.flip(v)

                @plsc.parallel_loop(0, H, unroll=8)
                def _rpos(r):
                    for c in range(NC):
                        v = in2[s][1, r, pl.ds(c * L, L)]
                        ob2[s][1, r, pl.ds((NC - 1 - c) * L, L)] = jnp.flip(v)

                start_out(s, p)

                @pl.when(p + NSLOT < p_hi)
                def _():
                    start_in(s, p + NSLOT)

    for s in range(NSLOT):
        wait_out(s, p_lo + s)


_sc_call = pl.kernel(
    _sc_body,
    out_type=(
        jax.ShapeDtypeStruct((B, P, H, W), jnp.float32),
        jax.ShapeDtypeStruct((B, P, 2, H, W), jnp.float32),
    ),
    mesh=_mesh,
    scratch_types=(
        [pltpu.VMEM((H, W), jnp.float32) for _ in range(NSLOT)],
        [pltpu.VMEM((2, H, W), jnp.float32) for _ in range(NSLOT)],
        [pltpu.VMEM((H, W), jnp.float32) for _ in range(NSLOT)],
        [pltpu.VMEM((2, H, W), jnp.float32) for _ in range(NSLOT)],
        [pltpu.SemaphoreType.DMA for _ in range(NSLOT)],
        [pltpu.SemaphoreType.DMA for _ in range(NSLOT)],
    ),
)


TCB = 8


def _tc_body(f1_ref, f2a_ref, f2b_ref, f2c_ref, o1_ref):
    idx1 = W - 1 - lax.broadcasted_iota(jnp.int32, (2, H, W), 2)
    revmap = {REV[0]: f2a_ref, REV[1]: f2b_ref, REV[2]: f2c_ref}
    for bi in range(TCB):
        for p in range(P):
            src = revmap[p][bi, 0] if p in REV else f1_ref[bi, PERM[p]]
            y = jnp.take_along_axis(src, idx1, axis=2)
            o1_ref[bi, p, 0] = -y[0]
            o1_ref[bi, p, 1] = y[1]


_tc_call = pl.pallas_call(
    _tc_body,
    grid=(B // TCB,),
    in_specs=[
        pl.BlockSpec((TCB, P, 2, H, W), lambda b: (b, 0, 0, 0, 0)),
        pl.BlockSpec((TCB, 1, 2, H, W), lambda b: (b, REV[0], 0, 0, 0)),
        pl.BlockSpec((TCB, 1, 2, H, W), lambda b: (b, REV[1], 0, 0, 0)),
        pl.BlockSpec((TCB, 1, 2, H, W), lambda b: (b, REV[2], 0, 0, 0)),
    ],
    out_specs=pl.BlockSpec((TCB, P, 2, H, W), lambda b: (b, 0, 0, 0, 0)),
    out_shape=jax.ShapeDtypeStruct((B, P, 2, H, W), jnp.float32),
)


@jax.jit
def kernel(field0, field1, field2):
    o0, o2 = _sc_call(field0, field1, field2)
    o1 = _tc_call(field1, field2, field2, field2)
    return (o0, o1, o2)
